# baseline (speedup 1.0000x reference)
import functools

import jax
import jax.numpy as jnp
from jax.experimental import pallas as pl

N = 4096
T = 16
BM = 256


def _proj_kernel(xi, xj1, xk, wi, wj1, wj2, wk, yi, y1, y2, yk):
    yi[...] = jnp.dot(xi[...], wi[...], preferred_element_type=jnp.float32)
    y1[...] = jnp.dot(xj1[...], wj1[...], preferred_element_type=jnp.float32)
    y2[...] = jnp.dot(xj1[...], wj2[...], preferred_element_type=jnp.float32)
    yk[...] = jnp.dot(xk[...], wk[...], preferred_element_type=jnp.float32)


def _merge_kernel(gi, aj, cj, gk, yi, y1, y2, yk, out):
    acc = jnp.dot(gi[...], yi[...], preferred_element_type=jnp.float32)
    acc += jnp.dot(aj[...], y1[...], preferred_element_type=jnp.float32)
    acc += jnp.dot(cj[...], y2[...], preferred_element_type=jnp.float32)
    acc += jnp.dot(gk[...], yk[...], preferred_element_type=jnp.float32)
    out[...] = jnp.maximum(acc, 0.0)


@jax.jit
def kernel(xi, xj1, xj2, xk, Gi2j, Adj2j, coAdj2j, Gk2j, W_i, W_j1, W_j2, W_k):
    del xj2

    y_shape = jax.ShapeDtypeStruct((N, T), jnp.float32)
    yi, y1, y2, yk = pl.pallas_call(
        _proj_kernel,
        out_shape=(y_shape, y_shape, y_shape, y_shape),
    )(xi, xj1, xk, W_i, W_j1, W_j2, W_k)

    grid = (N // BM,)
    row_spec = pl.BlockSpec((BM, N), lambda i: (i, 0))
    full_spec = pl.BlockSpec((N, T), lambda i: (0, 0))
    out = pl.pallas_call(
        _merge_kernel,
        grid=grid,
        in_specs=[row_spec, row_spec, row_spec, row_spec,
                  full_spec, full_spec, full_spec, full_spec],
        out_specs=pl.BlockSpec((BM, T), lambda i: (i, 0)),
        out_shape=jax.ShapeDtypeStruct((N, T), jnp.float32),
    )(Gi2j, Adj2j, coAdj2j, Gk2j, yi, y1, y2, yk)
    return out


# speedup vs baseline: 1.0108x; 1.0108x over previous
import functools

import jax
import jax.numpy as jnp
from jax.experimental import pallas as pl

N = 4096
T = 16
BM = 256


def _proj_kernel(xi, xj1, xk, wi, wj1, wj2, wk, yi, y1, y2, yk):
    yi[...] = jnp.dot(
        xi[...], wi[...], preferred_element_type=jnp.float32
    ).astype(jnp.bfloat16)
    y1[...] = jnp.dot(
        xj1[...], wj1[...], preferred_element_type=jnp.float32
    ).astype(jnp.bfloat16)
    y2[...] = jnp.dot(
        xj1[...], wj2[...], preferred_element_type=jnp.float32
    ).astype(jnp.bfloat16)
    yk[...] = jnp.dot(
        xk[...], wk[...], preferred_element_type=jnp.float32
    ).astype(jnp.bfloat16)


def _merge_kernel(gi, aj, cj, gk, yi, y1, y2, yk, out):
    bf = jnp.bfloat16
    acc = jnp.dot(gi[...].astype(bf), yi[...], preferred_element_type=jnp.float32)
    acc += jnp.dot(aj[...].astype(bf), y1[...], preferred_element_type=jnp.float32)
    acc += jnp.dot(cj[...].astype(bf), y2[...], preferred_element_type=jnp.float32)
    acc += jnp.dot(gk[...].astype(bf), yk[...], preferred_element_type=jnp.float32)
    out[...] = jnp.maximum(acc, 0.0)


@jax.jit
def kernel(xi, xj1, xj2, xk, Gi2j, Adj2j, coAdj2j, Gk2j, W_i, W_j1, W_j2, W_k):
    del xj2

    y_shape = jax.ShapeDtypeStruct((N, T), jnp.bfloat16)
    yi, y1, y2, yk = pl.pallas_call(
        _proj_kernel,
        out_shape=(y_shape, y_shape, y_shape, y_shape),
    )(xi, xj1, xk, W_i, W_j1, W_j2, W_k)

    grid = (N // BM,)
    row_spec = pl.BlockSpec((BM, N), lambda i: (i, 0))
    full_spec = pl.BlockSpec((N, T), lambda i: (0, 0))
    out = pl.pallas_call(
        _merge_kernel,
        grid=grid,
        in_specs=[row_spec, row_spec, row_spec, row_spec,
                  full_spec, full_spec, full_spec, full_spec],
        out_specs=pl.BlockSpec((BM, T), lambda i: (i, 0)),
        out_shape=jax.ShapeDtypeStruct((N, T), jnp.float32),
    )(Gi2j, Adj2j, coAdj2j, Gk2j, yi, y1, y2, yk)
    return out
